# baseline (speedup 1.0000x reference)
import functools

import numpy as np
import jax
import jax.numpy as jnp
from jax import lax
from jax.experimental import pallas as pl
from jax.experimental.pallas import tpu as pltpu
from jax.experimental.pallas import tpu_sc as plsc

_VOCAB = 1_000_000
_D = 64
_B = 4096
_L = 50
_BL = _B * _L
_TOTAL = _BL * _D
_NROWS = _TOTAL // 128

_MAG = np.float32(5.0) / np.sqrt(np.float32(_L * _D))

_KS0 = np.uint32(0)
_KS1 = np.uint32(42)
_KS2 = _KS0 ^ _KS1 ^ np.uint32(0x1BD11BDA)


def _rotl(x, r):
    return (x << np.uint32(r)) | (x >> np.uint32(32 - r))


def _tf_rounds(x0, x1, rots):
    for r in rots:
        x0 = x0 + x1
        x1 = _rotl(x1, r)
        x1 = x1 ^ x0
    return x0, x1


def _threefry_bits(i):
    r0 = (13, 15, 26, 6)
    r1 = (17, 29, 16, 24)
    x0 = jnp.zeros_like(i) + _KS0
    x1 = i + _KS1
    x0, x1 = _tf_rounds(x0, x1, r0)
    x0 = x0 + _KS1
    x1 = x1 + _KS2 + np.uint32(1)
    x0, x1 = _tf_rounds(x0, x1, r1)
    x0 = x0 + _KS2
    x1 = x1 + _KS0 + np.uint32(2)
    x0, x1 = _tf_rounds(x0, x1, r0)
    x0 = x0 + _KS0
    x1 = x1 + _KS1 + np.uint32(3)
    x0, x1 = _tf_rounds(x0, x1, r1)
    x0 = x0 + _KS1
    x1 = x1 + _KS2 + np.uint32(4)
    x0, x1 = _tf_rounds(x0, x1, r0)
    x0 = x0 + _KS2
    x1 = x1 + _KS0 + np.uint32(5)
    return x0 ^ x1


_TC = 2048


def _xpose_body(t_ref, o_ref):
    x = t_ref[...]
    d = lax.broadcasted_iota(jnp.int32, (64, 64), 0)
    j = lax.broadcasted_iota(jnp.int32, (64, 64), 1)
    eye = jnp.where(d == j, np.float32(1.0), np.float32(0.0))
    xt = lax.dot_general(
        x, eye, (((0,), (0,)), ((), ())), preferred_element_type=jnp.float32
    )
    o_ref[...] = jnp.concatenate((xt, xt), axis=1)


def _xpose(tT):
    return pl.pallas_call(
        _xpose_body,
        grid=(pl.cdiv(_VOCAB, _TC),),
        in_specs=[pl.BlockSpec((64, _TC), lambda i: (0, i))],
        out_specs=pl.BlockSpec((_TC, 128), lambda i: (i, 0)),
        out_shape=jax.ShapeDtypeStruct((_VOCAB, 128), jnp.float32),
    )(tT)


_BLKR = 1024


def _noise_body(o_ref):
    blk = pl.program_id(0)
    base = (blk * (_BLKR * 128)).astype(jnp.uint32)
    row = lax.broadcasted_iota(jnp.uint32, (_BLKR, 128), 0)
    col = lax.broadcasted_iota(jnp.uint32, (_BLKR, 128), 1)
    i = base + row * np.uint32(128) + col
    bits = _threefry_bits(i)
    fb = (bits >> np.uint32(9)) | np.uint32(0x3F800000)
    u = lax.bitcast_convert_type(fb, jnp.float32) - np.float32(1.0)
    n = jnp.maximum(np.float32(-1.0), u * np.float32(2.0) + np.float32(-1.0))
    o_ref[...] = n * _MAG


def _noise():
    return pl.pallas_call(
        _noise_body,
        grid=(_NROWS // _BLKR,),
        in_specs=[],
        out_specs=pl.BlockSpec((_BLKR, 128), lambda i: (i, 0)),
        out_shape=jax.ShapeDtypeStruct((_NROWS, 128), jnp.float32),
    )()


_AR = 2048


def _add_body(g_ref, n_ref, o_ref):
    o_ref[...] = g_ref[...] + n_ref[...]


def _add(g2, nz):
    return pl.pallas_call(
        _add_body,
        grid=(_NROWS // _AR,),
        in_specs=[
            pl.BlockSpec((_AR, 128), lambda i: (i, 0)),
            pl.BlockSpec((_AR, 128), lambda i: (i, 0)),
        ],
        out_specs=pl.BlockSpec((_AR, 128), lambda i: (i, 0)),
        out_shape=jax.ShapeDtypeStruct((_NROWS, 128), jnp.float32),
    )(g2, nz)


_info = plsc.get_sparse_core_info()
_NC, _NS = _info.num_cores, _info.num_subcores
_NW = _NC * _NS
_RPW = _BL // _NW
_NSUB = 5
_CH = _NSUB * 128
_NCHUNK = _RPW // _CH


def _gather_body(idx_hbm, table_hbm, out_hbm, idx_v, rows_v, sem):
    wid = lax.axis_index("s") * _NC + lax.axis_index("c")

    def chunk(ci, carry):
        r0 = wid * _RPW + ci * _CH
        pltpu.sync_copy(idx_hbm.at[pl.ds(r0, _CH)], idx_v)
        handles = [
            pltpu.async_copy(
                table_hbm.at[idx_v.at[pl.ds(j * 128, 128)]],
                rows_v.at[pl.ds(j * 128, 128)],
                sem,
            )
            for j in range(_NSUB)
        ]
        for h in handles:
            h.wait()
        pltpu.sync_copy(rows_v, out_hbm.at[pl.ds(r0, _CH)])
        return carry

    lax.fori_loop(0, _NCHUNK, chunk, 0)


_gather = functools.partial(
    pl.kernel,
    mesh=plsc.VectorSubcoreMesh(core_axis_name="c", subcore_axis_name="s"),
    out_type=jax.ShapeDtypeStruct((_BL, _D), jnp.float32),
    scratch_types=[
        pltpu.VMEM((_CH,), jnp.int32),
        pltpu.VMEM((_CH, _D), jnp.float32),
        pltpu.SemaphoreType.DMA,
    ],
    compiler_params=pltpu.CompilerParams(use_tc_tiling_on_sc=False),
)(_gather_body)


def kernel(input_ids, table):
    t2 = _xpose(table.T)
    t3 = t2.reshape(2 * _VOCAB, _D)
    idx = input_ids.reshape(_BL) * 2
    g = _gather(idx, t3)
    nz = _noise()
    out = _add(g.reshape(_NROWS, 128), nz)
    return out.reshape(_B, _L, _D)


# speedup vs baseline: 1.2693x; 1.2693x over previous
import functools

import numpy as np
import jax
import jax.numpy as jnp
from jax import lax
from jax.experimental import pallas as pl
from jax.experimental.pallas import tpu as pltpu
from jax.experimental.pallas import tpu_sc as plsc

_VOCAB = 1_000_000
_D = 64
_B = 4096
_L = 50
_BL = _B * _L
_TOTAL = _BL * _D
_NROWS = _TOTAL // 128

_MAG = np.float32(5.0) / np.sqrt(np.float32(_L * _D))

_KS0 = np.uint32(0)
_KS1 = np.uint32(42)
_KS2 = _KS0 ^ _KS1 ^ np.uint32(0x1BD11BDA)


def _rotl(x, r):
    return (x << np.uint32(r)) | (x >> np.uint32(32 - r))


def _tf_rounds(x0, x1, rots):
    for r in rots:
        x0 = x0 + x1
        x1 = _rotl(x1, r)
        x1 = x1 ^ x0
    return x0, x1


def _threefry_bits(i):
    r0 = (13, 15, 26, 6)
    r1 = (17, 29, 16, 24)
    x0 = jnp.zeros_like(i) + _KS0
    x1 = i + _KS1
    x0, x1 = _tf_rounds(x0, x1, r0)
    x0 = x0 + _KS1
    x1 = x1 + _KS2 + np.uint32(1)
    x0, x1 = _tf_rounds(x0, x1, r1)
    x0 = x0 + _KS2
    x1 = x1 + _KS0 + np.uint32(2)
    x0, x1 = _tf_rounds(x0, x1, r0)
    x0 = x0 + _KS0
    x1 = x1 + _KS1 + np.uint32(3)
    x0, x1 = _tf_rounds(x0, x1, r1)
    x0 = x0 + _KS1
    x1 = x1 + _KS2 + np.uint32(4)
    x0, x1 = _tf_rounds(x0, x1, r0)
    x0 = x0 + _KS2
    x1 = x1 + _KS0 + np.uint32(5)
    return x0 ^ x1


_TC = 8192


def _xpose_body(t_ref, o_ref):
    x = t_ref[...]
    d = lax.broadcasted_iota(jnp.int32, (64, 64), 0)
    j = lax.broadcasted_iota(jnp.int32, (64, 64), 1)
    eye = jnp.where(d == j, np.float32(1.0), np.float32(0.0))
    xt = lax.dot_general(
        x, eye, (((0,), (0,)), ((), ())), preferred_element_type=jnp.float32
    )
    o_ref[...] = jnp.concatenate((xt, xt), axis=1)


def _xpose(tT):
    return pl.pallas_call(
        _xpose_body,
        grid=(pl.cdiv(_VOCAB, _TC),),
        in_specs=[pl.BlockSpec((64, _TC), lambda i: (0, i))],
        out_specs=pl.BlockSpec((_TC, 128), lambda i: (i, 0)),
        out_shape=jax.ShapeDtypeStruct((_VOCAB, 128), jnp.float32),
    )(tT)


_BLKR = 1024


def _noise_body(o_ref):
    blk = pl.program_id(0)
    base = (blk * (_BLKR * 128)).astype(jnp.uint32)
    row = lax.broadcasted_iota(jnp.uint32, (_BLKR, 128), 0)
    col = lax.broadcasted_iota(jnp.uint32, (_BLKR, 128), 1)
    i = base + row * np.uint32(128) + col
    bits = _threefry_bits(i)
    fb = (bits >> np.uint32(9)) | np.uint32(0x3F800000)
    u = lax.bitcast_convert_type(fb, jnp.float32) - np.float32(1.0)
    n = jnp.maximum(np.float32(-1.0), u * np.float32(2.0) + np.float32(-1.0))
    o_ref[...] = n * _MAG


def _noise():
    return pl.pallas_call(
        _noise_body,
        grid=(_NROWS // _BLKR,),
        in_specs=[],
        out_specs=pl.BlockSpec((_BLKR, 128), lambda i: (i, 0)),
        out_shape=jax.ShapeDtypeStruct((_NROWS, 128), jnp.float32),
    )()


_AR = 2048


def _add_body(g_ref, n_ref, o_ref):
    o_ref[...] = g_ref[...] + n_ref[...]


def _add(g2, nz):
    return pl.pallas_call(
        _add_body,
        grid=(_NROWS // _AR,),
        in_specs=[
            pl.BlockSpec((_AR, 128), lambda i: (i, 0)),
            pl.BlockSpec((_AR, 128), lambda i: (i, 0)),
        ],
        out_specs=pl.BlockSpec((_AR, 128), lambda i: (i, 0)),
        out_shape=jax.ShapeDtypeStruct((_NROWS, 128), jnp.float32),
    )(g2, nz)


_info = plsc.get_sparse_core_info()
_NC, _NS = _info.num_cores, _info.num_subcores
_NW = _NC * _NS
_RPW = _BL // _NW
_NSUB = 5
_CH = _NSUB * 128
_NCHUNK = _RPW // _CH


def _gather_body(idx_hbm, table_hbm, out_hbm, idx_v, rows_v, sem):
    wid = lax.axis_index("s") * _NC + lax.axis_index("c")

    def chunk(ci, carry):
        r0 = wid * _RPW + ci * _CH
        pltpu.sync_copy(idx_hbm.at[pl.ds(r0, _CH)], idx_v)
        handles = [
            pltpu.async_copy(
                table_hbm.at[idx_v.at[pl.ds(j * 128, 128)]],
                rows_v.at[pl.ds(j * 128, 128)],
                sem,
            )
            for j in range(_NSUB)
        ]
        for h in handles:
            h.wait()
        pltpu.sync_copy(rows_v, out_hbm.at[pl.ds(r0, _CH)])
        return carry

    lax.fori_loop(0, _NCHUNK, chunk, 0)


_gather = functools.partial(
    pl.kernel,
    mesh=plsc.VectorSubcoreMesh(core_axis_name="c", subcore_axis_name="s"),
    out_type=jax.ShapeDtypeStruct((_BL, _D), jnp.float32),
    scratch_types=[
        pltpu.VMEM((_CH,), jnp.int32),
        pltpu.VMEM((_CH, _D), jnp.float32),
        pltpu.SemaphoreType.DMA,
    ],
    compiler_params=pltpu.CompilerParams(use_tc_tiling_on_sc=False),
)(_gather_body)


def kernel(input_ids, table):
    t2 = _xpose(table.T)
    t3 = t2.reshape(2 * _VOCAB, _D)
    idx = input_ids.reshape(_BL) * 2
    g = _gather(idx, t3)
    nz = _noise()
    out = _add(g.reshape(_NROWS, 128), nz)
    return out.reshape(_B, _L, _D)


# speedup vs baseline: 1.3308x; 1.0485x over previous
import functools

import numpy as np
import jax
import jax.numpy as jnp
from jax import lax
from jax.experimental import pallas as pl
from jax.experimental.pallas import tpu as pltpu
from jax.experimental.pallas import tpu_sc as plsc

_VOCAB = 1_000_000
_D = 64
_B = 4096
_L = 50
_BL = _B * _L
_TOTAL = _BL * _D
_NROWS = _TOTAL // 128

_MAG = np.float32(5.0) / np.sqrt(np.float32(_L * _D))

_KS0 = np.uint32(0)
_KS1 = np.uint32(42)
_KS2 = _KS0 ^ _KS1 ^ np.uint32(0x1BD11BDA)


def _rotl(x, r):
    return (x << np.uint32(r)) | (x >> np.uint32(32 - r))


def _tf_rounds(x0, x1, rots):
    for r in rots:
        x0 = x0 + x1
        x1 = _rotl(x1, r)
        x1 = x1 ^ x0
    return x0, x1


def _threefry_bits(i):
    r0 = (13, 15, 26, 6)
    r1 = (17, 29, 16, 24)
    x0 = jnp.zeros_like(i) + _KS0
    x1 = i + _KS1
    x0, x1 = _tf_rounds(x0, x1, r0)
    x0 = x0 + _KS1
    x1 = x1 + _KS2 + np.uint32(1)
    x0, x1 = _tf_rounds(x0, x1, r1)
    x0 = x0 + _KS2
    x1 = x1 + _KS0 + np.uint32(2)
    x0, x1 = _tf_rounds(x0, x1, r0)
    x0 = x0 + _KS0
    x1 = x1 + _KS1 + np.uint32(3)
    x0, x1 = _tf_rounds(x0, x1, r1)
    x0 = x0 + _KS1
    x1 = x1 + _KS2 + np.uint32(4)
    x0, x1 = _tf_rounds(x0, x1, r0)
    x0 = x0 + _KS2
    x1 = x1 + _KS0 + np.uint32(5)
    return x0 ^ x1


_TC = 16384


def _xpose_body(t_ref, o_ref):
    x = t_ref[...]
    d = lax.broadcasted_iota(jnp.int32, (64, 64), 0)
    j = lax.broadcasted_iota(jnp.int32, (64, 64), 1)
    eye = jnp.where(d == j, np.float32(1.0), np.float32(0.0))
    xt = lax.dot_general(
        x, eye, (((0,), (0,)), ((), ())), preferred_element_type=jnp.float32
    )
    o_ref[...] = jnp.concatenate((xt, xt), axis=1)


def _xpose(tT):
    return pl.pallas_call(
        _xpose_body,
        grid=(pl.cdiv(_VOCAB, _TC),),
        in_specs=[pl.BlockSpec((64, _TC), lambda i: (0, i))],
        out_specs=pl.BlockSpec((_TC, 128), lambda i: (i, 0)),
        out_shape=jax.ShapeDtypeStruct((_VOCAB, 128), jnp.float32),
    )(tT)


_BLKR = 1024


def _noise_body(o_ref):
    blk = pl.program_id(0)
    base = (blk * (_BLKR * 128)).astype(jnp.uint32)
    row = lax.broadcasted_iota(jnp.uint32, (_BLKR, 128), 0)
    col = lax.broadcasted_iota(jnp.uint32, (_BLKR, 128), 1)
    i = base + row * np.uint32(128) + col
    bits = _threefry_bits(i)
    fb = (bits >> np.uint32(9)) | np.uint32(0x3F800000)
    u = lax.bitcast_convert_type(fb, jnp.float32) - np.float32(1.0)
    n = jnp.maximum(np.float32(-1.0), u * np.float32(2.0) + np.float32(-1.0))
    o_ref[...] = n * _MAG


def _noise():
    return pl.pallas_call(
        _noise_body,
        grid=(_NROWS // _BLKR,),
        in_specs=[],
        out_specs=pl.BlockSpec((_BLKR, 128), lambda i: (i, 0)),
        out_shape=jax.ShapeDtypeStruct((_NROWS, 128), jnp.float32),
    )()


_AR = 2048


def _add_body(g_ref, n_ref, o_ref):
    o_ref[...] = g_ref[...] + n_ref[...]


def _add(g2, nz):
    return pl.pallas_call(
        _add_body,
        grid=(_NROWS // _AR,),
        in_specs=[
            pl.BlockSpec((_AR, 128), lambda i: (i, 0)),
            pl.BlockSpec((_AR, 128), lambda i: (i, 0)),
        ],
        out_specs=pl.BlockSpec((_AR, 128), lambda i: (i, 0)),
        out_shape=jax.ShapeDtypeStruct((_NROWS, 128), jnp.float32),
    )(g2, nz)


_info = plsc.get_sparse_core_info()
_NC, _NS = _info.num_cores, _info.num_subcores
_NW = _NC * _NS
_RPW = _BL // _NW
_NSUB = 5
_CH = _NSUB * 128
_NCHUNK = _RPW // _CH


def _gather_body(idx_hbm, table_hbm, out_hbm, idx_v, rows_v, sem):
    wid = lax.axis_index("s") * _NC + lax.axis_index("c")

    def chunk(ci, carry):
        r0 = wid * _RPW + ci * _CH
        pltpu.sync_copy(idx_hbm.at[pl.ds(r0, _CH)], idx_v)
        handles = [
            pltpu.async_copy(
                table_hbm.at[idx_v.at[pl.ds(j * 128, 128)]],
                rows_v.at[pl.ds(j * 128, 128)],
                sem,
            )
            for j in range(_NSUB)
        ]
        for h in handles:
            h.wait()
        pltpu.sync_copy(rows_v, out_hbm.at[pl.ds(r0, _CH)])
        return carry

    lax.fori_loop(0, _NCHUNK, chunk, 0)


_gather = functools.partial(
    pl.kernel,
    mesh=plsc.VectorSubcoreMesh(core_axis_name="c", subcore_axis_name="s"),
    out_type=jax.ShapeDtypeStruct((_BL, _D), jnp.float32),
    scratch_types=[
        pltpu.VMEM((_CH,), jnp.int32),
        pltpu.VMEM((_CH, _D), jnp.float32),
        pltpu.SemaphoreType.DMA,
    ],
    compiler_params=pltpu.CompilerParams(use_tc_tiling_on_sc=False),
)(_gather_body)


def kernel(input_ids, table):
    t2 = _xpose(table.T)
    t3 = t2.reshape(2 * _VOCAB, _D)
    idx = input_ids.reshape(_BL) * 2
    g = _gather(idx, t3)
    nz = _noise()
    out = _add(g.reshape(_NROWS, 128), nz)
    return out.reshape(_B, _L, _D)


# speedup vs baseline: 1.3511x; 1.0152x over previous
import functools

import numpy as np
import jax
import jax.numpy as jnp
from jax import lax
from jax.experimental import pallas as pl
from jax.experimental.pallas import tpu as pltpu
from jax.experimental.pallas import tpu_sc as plsc

_VOCAB = 1_000_000
_D = 64
_B = 4096
_L = 50
_BL = _B * _L
_TOTAL = _BL * _D
_NROWS = _TOTAL // 128

_MAG = np.float32(5.0) / np.sqrt(np.float32(_L * _D))

_KS0 = np.uint32(0)
_KS1 = np.uint32(42)
_KS2 = _KS0 ^ _KS1 ^ np.uint32(0x1BD11BDA)


def _rotl(x, r):
    return (x << np.uint32(r)) | (x >> np.uint32(32 - r))


def _tf_rounds(x0, x1, rots):
    for r in rots:
        x0 = x0 + x1
        x1 = _rotl(x1, r)
        x1 = x1 ^ x0
    return x0, x1


def _threefry_bits(i):
    r0 = (13, 15, 26, 6)
    r1 = (17, 29, 16, 24)
    x0 = jnp.zeros_like(i) + _KS0
    x1 = i + _KS1
    x0, x1 = _tf_rounds(x0, x1, r0)
    x0 = x0 + _KS1
    x1 = x1 + _KS2 + np.uint32(1)
    x0, x1 = _tf_rounds(x0, x1, r1)
    x0 = x0 + _KS2
    x1 = x1 + _KS0 + np.uint32(2)
    x0, x1 = _tf_rounds(x0, x1, r0)
    x0 = x0 + _KS0
    x1 = x1 + _KS1 + np.uint32(3)
    x0, x1 = _tf_rounds(x0, x1, r1)
    x0 = x0 + _KS1
    x1 = x1 + _KS2 + np.uint32(4)
    x0, x1 = _tf_rounds(x0, x1, r0)
    x0 = x0 + _KS2
    x1 = x1 + _KS0 + np.uint32(5)
    return x0 ^ x1


_TC = 24576


def _xpose_body(t_ref, o_ref):
    x = t_ref[...]
    d = lax.broadcasted_iota(jnp.int32, (64, 64), 0)
    j = lax.broadcasted_iota(jnp.int32, (64, 64), 1)
    eye = jnp.where(d == j, np.float32(1.0), np.float32(0.0))
    xt = lax.dot_general(
        x, eye, (((0,), (0,)), ((), ())), preferred_element_type=jnp.float32
    )
    o_ref[...] = jnp.concatenate((xt, xt), axis=1)


def _xpose(tT):
    return pl.pallas_call(
        _xpose_body,
        grid=(pl.cdiv(_VOCAB, _TC),),
        in_specs=[pl.BlockSpec((64, _TC), lambda i: (0, i))],
        out_specs=pl.BlockSpec((_TC, 128), lambda i: (i, 0)),
        out_shape=jax.ShapeDtypeStruct((_VOCAB, 128), jnp.float32),
    )(tT)


_BLKR = 1024


def _noise_body(o_ref):
    blk = pl.program_id(0)
    base = (blk * (_BLKR * 128)).astype(jnp.uint32)
    row = lax.broadcasted_iota(jnp.uint32, (_BLKR, 128), 0)
    col = lax.broadcasted_iota(jnp.uint32, (_BLKR, 128), 1)
    i = base + row * np.uint32(128) + col
    bits = _threefry_bits(i)
    fb = (bits >> np.uint32(9)) | np.uint32(0x3F800000)
    u = lax.bitcast_convert_type(fb, jnp.float32) - np.float32(1.0)
    n = jnp.maximum(np.float32(-1.0), u * np.float32(2.0) + np.float32(-1.0))
    o_ref[...] = n * _MAG


def _noise():
    return pl.pallas_call(
        _noise_body,
        grid=(_NROWS // _BLKR,),
        in_specs=[],
        out_specs=pl.BlockSpec((_BLKR, 128), lambda i: (i, 0)),
        out_shape=jax.ShapeDtypeStruct((_NROWS, 128), jnp.float32),
    )()


_AR = 2048


def _add_body(g_ref, n_ref, o_ref):
    o_ref[...] = g_ref[...] + n_ref[...]


def _add(g2, nz):
    return pl.pallas_call(
        _add_body,
        grid=(_NROWS // _AR,),
        in_specs=[
            pl.BlockSpec((_AR, 128), lambda i: (i, 0)),
            pl.BlockSpec((_AR, 128), lambda i: (i, 0)),
        ],
        out_specs=pl.BlockSpec((_AR, 128), lambda i: (i, 0)),
        out_shape=jax.ShapeDtypeStruct((_NROWS, 128), jnp.float32),
    )(g2, nz)


_info = plsc.get_sparse_core_info()
_NC, _NS = _info.num_cores, _info.num_subcores
_NW = _NC * _NS
_RPW = _BL // _NW
_NSUB = 5
_CH = _NSUB * 128
_NCHUNK = _RPW // _CH


def _gather_body(idx_hbm, table_hbm, out_hbm, idx_v, rows_v, sem):
    wid = lax.axis_index("s") * _NC + lax.axis_index("c")

    def chunk(ci, carry):
        r0 = wid * _RPW + ci * _CH
        pltpu.sync_copy(idx_hbm.at[pl.ds(r0, _CH)], idx_v)
        handles = [
            pltpu.async_copy(
                table_hbm.at[idx_v.at[pl.ds(j * 128, 128)]],
                rows_v.at[pl.ds(j * 128, 128)],
                sem,
            )
            for j in range(_NSUB)
        ]
        for h in handles:
            h.wait()
        pltpu.sync_copy(rows_v, out_hbm.at[pl.ds(r0, _CH)])
        return carry

    lax.fori_loop(0, _NCHUNK, chunk, 0)


_gather = functools.partial(
    pl.kernel,
    mesh=plsc.VectorSubcoreMesh(core_axis_name="c", subcore_axis_name="s"),
    out_type=jax.ShapeDtypeStruct((_BL, _D), jnp.float32),
    scratch_types=[
        pltpu.VMEM((_CH,), jnp.int32),
        pltpu.VMEM((_CH, _D), jnp.float32),
        pltpu.SemaphoreType.DMA,
    ],
    compiler_params=pltpu.CompilerParams(use_tc_tiling_on_sc=False),
)(_gather_body)


def kernel(input_ids, table):
    t2 = _xpose(table.T)
    t3 = t2.reshape(2 * _VOCAB, _D)
    idx = input_ids.reshape(_BL) * 2
    g = _gather(idx, t3)
    nz = _noise()
    out = _add(g.reshape(_NROWS, 128), nz)
    return out.reshape(_B, _L, _D)
